# baseline (speedup 1.0000x reference)
import functools

import jax
import jax.numpy as jnp
from jax import lax
from jax.experimental import pallas as pl
from jax.experimental.pallas import tpu as pltpu
from jax.experimental.pallas import tpu_sc as plsc

_ND = 25000
_NT = 25000
_N = _ND + _NT
_E = 400000
_H = 128
_L = 2

_NC = 2
_NS = 16
_NW = _NC * _NS

_E_PAD = 409600
_ACH = _E_PAD // _NW
_SUB = 2560
_TCH = _E_PAD // _NS

_N_A = 50016
_RSZ = 9472
_NR = 6
_N_X = _RSZ * _NR
_ACC_ROWS = 9728
_TRASH = _RSZ
_FR = _RSZ // _NS
_ZR = 76
_K = 256
_CMAX = 2832

_MESH = dict(core_axis_name="c", subcore_axis_name="s", num_cores=_NC,
             num_subcores=_NS)


def _proj_body(x_ref, w_ref, b_ref, o_ref):
  o_ref[...] = (
      jnp.dot(x_ref[...], w_ref[...], preferred_element_type=jnp.float32,
              precision=lax.Precision.HIGHEST)
      + b_ref[...][None, :]
  )


def _proj(x, w, b):
  rows, k = x.shape
  rb = 5000
  return pl.pallas_call(
      _proj_body,
      grid=(rows // rb,),
      in_specs=[
          pl.BlockSpec((rb, k), lambda i: (i, 0)),
          pl.BlockSpec((k, _H), lambda i: (0, 0)),
          pl.BlockSpec((_H,), lambda i: (0,)),
      ],
      out_specs=pl.BlockSpec((rb, _H), lambda i: (i, 0)),
      out_shape=jax.ShapeDtypeStruct((rows, _H), jnp.float32),
  )(x, w, b)


def _hw_body(x_ref, w_ref, a_ref, hw_ref, al_ref):
  h = jnp.dot(x_ref[...], w_ref[0], preferred_element_type=jnp.float32,
              precision=lax.Precision.HIGHEST)
  hw_ref[0] = h
  al_ref[0] = jnp.dot(h, a_ref[0], preferred_element_type=jnp.float32,
                      precision=lax.Precision.HIGHEST)


def _hw_alpha(x, w3, a3):
  rb = 5000
  return pl.pallas_call(
      _hw_body,
      grid=(3, _N // rb),
      in_specs=[
          pl.BlockSpec((rb, _H), lambda t, i: (i, 0)),
          pl.BlockSpec((1, _H, _H), lambda t, i: (t, 0, 0)),
          pl.BlockSpec((1, _H, 2), lambda t, i: (t, 0, 0)),
      ],
      out_specs=[
          pl.BlockSpec((1, rb, _H), lambda t, i: (t, i, 0)),
          pl.BlockSpec((1, rb, 2), lambda t, i: (t, i, 0)),
      ],
      out_shape=[
          jax.ShapeDtypeStruct((3, _N, _H), jnp.float32),
          jax.ShapeDtypeStruct((3, _N, 2), jnp.float32),
      ],
  )(x, w3, a3)


def _term_body(ea_ref, ae_ref, o_ref):
  o_ref[...] = jnp.dot(ea_ref[...], ae_ref[...],
                       preferred_element_type=jnp.float32,
                       precision=lax.Precision.HIGHEST)


def _edge_terms(edge_attr, a_edge):
  eb = 8000
  de = edge_attr.shape[1]
  return pl.pallas_call(
      _term_body,
      grid=(_E // eb,),
      in_specs=[
          pl.BlockSpec((eb, de), lambda i: (i, 0)),
          pl.BlockSpec((de, _L), lambda i: (0, 0)),
      ],
      out_specs=pl.BlockSpec((eb, _L), lambda i: (i, 0)),
      out_shape=jax.ShapeDtypeStruct((_E, _L), jnp.float32),
  )(edge_attr, a_edge.T)


def _make_phase_a(has_term):
  mesh = plsc.VectorSubcoreMesh(**_MESH)
  nsub = _ACH // _SUB
  nv = _SUB // 16

  scratch = [
      pltpu.VMEM_SHARED((_N_A,), jnp.float32),
      pltpu.VMEM_SHARED((_N_A,), jnp.float32),
      pltpu.VMEM((_N_A,), jnp.float32),
      pltpu.VMEM((_N_A,), jnp.float32),
      pltpu.VMEM((16,), jnp.float32),
      pltpu.VMEM((_SUB,), jnp.int32),
      pltpu.VMEM((_SUB,), jnp.int32),
      pltpu.VMEM((_SUB,), jnp.float32),
      pltpu.VMEM((_SUB,), jnp.float32),
  ]

  def body(as_hbm, ad_hbm, m_hbm, src_hbm, dst_hbm, term_hbm, e_hbm,
           as_sh, ad_sh, as_v, ad_v, mv, sv, dv, tv, ev):
    cid = lax.axis_index("c")
    sid = lax.axis_index("s")
    wid = sid * _NC + cid

    @pl.when(sid == 0)
    def _stage():
      pltpu.sync_copy(as_hbm, as_sh)
      pltpu.sync_copy(ad_hbm, ad_sh)

    pltpu.sync_copy(m_hbm, mv)
    plsc.subcore_barrier()
    pltpu.sync_copy(as_sh, as_v)
    pltpu.sync_copy(ad_sh, ad_v)
    m16 = mv[...]

    def sub(su, _):
      base = wid * _ACH + su * _SUB
      pltpu.sync_copy(src_hbm.at[pl.ds(base, _SUB)], sv)
      pltpu.sync_copy(dst_hbm.at[pl.ds(base, _SUB)], dv)
      if has_term:
        pltpu.sync_copy(term_hbm.at[pl.ds(base, _SUB)], tv)

      def vec(v, _):
        s16 = sv[pl.ds(v * 16, 16)]
        d16 = dv[pl.ds(v * 16, 16)]
        lg = plsc.load_gather(as_v, [s16]) + plsc.load_gather(ad_v, [d16])
        if has_term:
          lg = lg + tv[pl.ds(v * 16, 16)]
        lr = jnp.where(lg >= 0.0, lg, lg * jnp.float32(0.2))
        ev[pl.ds(v * 16, 16)] = jnp.exp(lr - m16)
        return 0

      lax.fori_loop(0, nv, vec, 0)
      pltpu.sync_copy(ev, e_hbm.at[pl.ds(base, _SUB)])
      return 0

    lax.fori_loop(0, nsub, sub, 0)

  return pl.kernel(
      body,
      out_type=jax.ShapeDtypeStruct((_E_PAD,), jnp.float32),
      mesh=mesh,
      scratch_types=scratch,
      compiler_params=pltpu.CompilerParams(needs_layout_passes=False),
  )


_phase_a_term = _make_phase_a(True)
_phase_a_noterm = _make_phase_a(False)


def _make_phase_b():
  mesh = plsc.VectorSubcoreMesh(**_MESH)
  nsub = _TCH // _SUB
  nv = _SUB // 16

  scratch = [
      pltpu.VMEM_SHARED((_ACC_ROWS, _H), jnp.float32),
      pltpu.VMEM_SHARED((_ACC_ROWS,), jnp.float32),
      pltpu.VMEM((608,), jnp.float32),
      pltpu.VMEM((_SUB,), jnp.int32),
      pltpu.VMEM((_SUB,), jnp.int32),
      pltpu.VMEM((_SUB,), jnp.float32),
      pltpu.VMEM((_CMAX,), jnp.int32),
      pltpu.VMEM((_CMAX,), jnp.int32),
      pltpu.VMEM((_CMAX,), jnp.float32),
      pltpu.VMEM((_K,), jnp.int32),
      pltpu.VMEM((_K, _H), jnp.float32),
      pltpu.SemaphoreType.DMA,
  ]

  def body(hw0, hw1, hw2, s0, d0, e0, s1, d1, e1, s2, d2, e2,
           acc0, acc1, acc2, den0, den1, den2,
           acc_sh, den_sh, zden, sv, dv, evv, sC, dC, eC,
           dBlk, rows, gsem):
    cid = lax.axis_index("c")
    sid = lax.axis_index("s")
    z16f = jnp.zeros((16,), jnp.float32)
    z16i = jnp.zeros((16,), jnp.int32)
    t16i = jnp.full((16,), _TRASH, jnp.int32)

    for i in range(608 // 16):
      zden[pl.ds(i * 16, 16)] = z16f

    def flush_block(hw, off, n_idx_stage):
      for i in range(_K // 16):
        dBlk[pl.ds(i * 16, 16)] = dC[pl.ds(off + i * 16, 16)]
      pltpu.async_copy(hw.at[sC.at[pl.ds(off, _K)]], rows, gsem).wait()

      def scale(i2, _):
        for i in (i2 * 2, i2 * 2 + 1):
          ev = plsc.load_gather(eC, [jnp.full((16,), off + i, jnp.int32)])
          for j in range(_H // 16):
            rows[i, pl.ds(j * 16, 16)] = rows[i, pl.ds(j * 16, 16)] * ev
        return 0

      lax.fori_loop(0, _K // 2, scale, 0)

      pltpu.sync_copy(rows, acc_sh.at[dBlk], add=True)
      pltpu.sync_copy(eC.at[pl.ds(off, _K)], den_sh.at[dBlk], add=True)

    def one_type(lo, hw, ss, dd, ee, accT, denT):
      def zinit(i, _):
        for j in range(_H // 16):
          rows[i, pl.ds(j * 16, 16)] = z16f
        return 0

      lax.fori_loop(0, _ZR, zinit, 0)
      plsc.subcore_barrier()
      for z in range(608 // _ZR):
        pltpu.sync_copy(rows.at[pl.ds(0, _ZR), :],
                        acc_sh.at[pl.ds(sid * 608 + z * _ZR, _ZR), :])
      pltpu.sync_copy(zden, den_sh.at[pl.ds(sid * 608, 608)])
      plsc.subcore_barrier()

      def sub(su, cnt):
        base = sid * _TCH + su * _SUB
        pltpu.sync_copy(ss.at[pl.ds(base, _SUB)], sv)
        pltpu.sync_copy(dd.at[pl.ds(base, _SUB)], dv)
        pltpu.sync_copy(ee.at[pl.ds(base, _SUB)], evv)

        def vec(v, cnt):
          d16 = dv[pl.ds(v * 16, 16)]
          rel = d16 - lo
          msk = (rel >= 0) & (rel < _RSZ)
          plsc.store_compressed(sC.at[pl.ds(cnt, 16)],
                                sv[pl.ds(v * 16, 16)], mask=msk)
          plsc.store_compressed(dC.at[pl.ds(cnt, 16)], rel, mask=msk)
          plsc.store_compressed(eC.at[pl.ds(cnt, 16)],
                                evv[pl.ds(v * 16, 16)], mask=msk)
          return cnt + plsc.all_reduce_population_count(msk)[0]

        cnt = lax.fori_loop(0, nv, vec, cnt)

        nfull = lax.div(cnt, jnp.int32(_K))

        def blk(b, _):
          flush_block(hw, b * _K, 0)
          return 0

        lax.fori_loop(0, nfull, blk, 0)

        roff = nfull * _K
        for i in range(_K // 16):
          sC[pl.ds(i * 16, 16)] = sC[pl.ds(roff + i * 16, 16)]
          dC[pl.ds(i * 16, 16)] = dC[pl.ds(roff + i * 16, 16)]
          eC[pl.ds(i * 16, 16)] = eC[pl.ds(roff + i * 16, 16)]
        return cnt - roff

      cnt = lax.fori_loop(0, nsub, sub, jnp.int32(0))

      for i in range(_K // 16):
        sC[pl.ds(cnt + i * 16, 16)] = z16i
        dC[pl.ds(cnt + i * 16, 16)] = t16i
        eC[pl.ds(cnt + i * 16, 16)] = z16f

      @pl.when(cnt > 0)
      def _tail():
        flush_block(hw, 0, 0)

      plsc.subcore_barrier()

      pltpu.sync_copy(acc_sh.at[pl.ds(sid * _FR, _FR), :],
                      accT.at[pl.ds(lo + sid * _FR, _FR), :])

      pltpu.sync_copy(den_sh.at[pl.ds(sid * _FR, _FR)],
                      evv.at[pl.ds(0, _FR)])
      pltpu.sync_copy(evv.at[pl.ds(0, _FR)],
                      denT.at[pl.ds(lo + sid * _FR, _FR)])

    def one_range(r, _):
      lo = (cid * (_NR // _NC) + r) * _RSZ
      one_type(lo, hw0, s0, d0, e0, acc0, den0)
      one_type(lo, hw1, s1, d1, e1, acc1, den1)
      one_type(lo, hw2, s2, d2, e2, acc2, den2)
      return 0

    lax.fori_loop(0, _NR // _NC, one_range, 0)

  return pl.kernel(
      body,
      out_type=(
          jax.ShapeDtypeStruct((_N_X, _H), jnp.float32),
          jax.ShapeDtypeStruct((_N_X, _H), jnp.float32),
          jax.ShapeDtypeStruct((_N_X, _H), jnp.float32),
          jax.ShapeDtypeStruct((_N_X,), jnp.float32),
          jax.ShapeDtypeStruct((_N_X,), jnp.float32),
          jax.ShapeDtypeStruct((_N_X,), jnp.float32),
      ),
      mesh=mesh,
      scratch_types=scratch,
      compiler_params=pltpu.CompilerParams(needs_layout_passes=False),
  )


_phase_b = _make_phase_b()


def _fin_body(a0_ref, a1_ref, a2_ref, d0_ref, d1_ref, d2_ref, o_ref):
  acc = None
  for a_ref, d_ref in ((a0_ref, d0_ref), (a1_ref, d1_ref), (a2_ref, d2_ref)):
    o = a_ref[...] / (d_ref[...] + jnp.float32(1e-16))
    o = jnp.where(o > 0.0, o, jnp.exp(o) - jnp.float32(1.0))
    acc = o if acc is None else acc + o
  o_ref[...] = acc * jnp.float32(1.0 / 3.0)


def _finalize(a0, a1, a2, d0, d1, d2):
  rb = 3552
  return pl.pallas_call(
      _fin_body,
      grid=(_N_X // rb,),
      in_specs=[
          pl.BlockSpec((rb, _H), lambda i: (i, 0)),
          pl.BlockSpec((rb, _H), lambda i: (i, 0)),
          pl.BlockSpec((rb, _H), lambda i: (i, 0)),
          pl.BlockSpec((rb, 1), lambda i: (i, 0)),
          pl.BlockSpec((rb, 1), lambda i: (i, 0)),
          pl.BlockSpec((rb, 1), lambda i: (i, 0)),
      ],
      out_specs=pl.BlockSpec((rb, _H), lambda i: (i, 0)),
      out_shape=jax.ShapeDtypeStruct((_N_X, _H), jnp.float32),
  )(a0, a1, a2, d0[:, None], d1[:, None], d2[:, None])


def _pad_edges(ei):
  src = jnp.pad(ei[0], (0, _E_PAD - _E))
  dst = jnp.pad(ei[1], (0, _E_PAD - _E), constant_values=_N)
  return src, dst


@jax.jit
def _run(drug_x, target_x, edge_attr_dd, W_drug, b_drug, W_target, b_target,
         W_gat, a_src, a_dst, a_edge, edge_index_dd, edge_index_dt,
         edge_index_tt):
  xd = _proj(drug_x, W_drug, b_drug)
  xt = _proj(target_x, W_target, b_target)
  x = jnp.concatenate([xd, xt], axis=0)

  terms = _edge_terms(edge_attr_dd, a_edge)

  edges = [_pad_edges(edge_index_dd), _pad_edges(edge_index_dt),
           _pad_edges(edge_index_tt)]

  for l in range(_L):
    a3 = jnp.stack([a_src[l], a_dst[l]], axis=-1)
    hw, al = _hw_alpha(x, W_gat[l], a3)
    term = jnp.pad(terms[:, l], (0, _E_PAD - _E))
    es = []
    for t in range(3):
      a_s = jnp.pad(al[t, :, 0], (0, _N_A - _N))
      a_d = jnp.pad(al[t, :, 1], (0, _N_A - _N))
      m = jnp.max(al[t, :, 0]) + jnp.max(al[t, :, 1])
      if t == 0:
        m = m + jnp.max(terms[:, l])
      m = jnp.maximum(m, 0.0)
      m16 = jnp.full((16,), m, jnp.float32)
      src, dst = edges[t]
      if t == 0:
        e = _phase_a_term(a_s, a_d, m16, src, dst, term)
      else:
        e = _phase_a_noterm(a_s, a_d, m16, src, dst, term)
      es.append(e)

    a0, a1, a2, d0, d1, d2 = _phase_b(
        hw[0], hw[1], hw[2],
        edges[0][0], edges[0][1], es[0],
        edges[1][0], edges[1][1], es[1],
        edges[2][0], edges[2][1], es[2])
    xn = _finalize(a0, a1, a2, d0, d1, d2)
    x = xn[:_N]

  return x[:_ND], x[_ND:]


def kernel(drug_x, target_x, edge_attr_dd, W_drug, b_drug, W_target,
           b_target, W_gat, a_src, a_dst, a_edge, edge_index_dd,
           edge_index_dt, edge_index_tt):
  return _run(drug_x, target_x, edge_attr_dd, W_drug, b_drug, W_target,
              b_target, W_gat, a_src, a_dst, a_edge, edge_index_dd,
              edge_index_dt, edge_index_tt)


# speedup vs baseline: 1.0250x; 1.0250x over previous
import functools

import jax
import jax.numpy as jnp
from jax import lax
from jax.experimental import pallas as pl
from jax.experimental.pallas import tpu as pltpu
from jax.experimental.pallas import tpu_sc as plsc

_ND = 25000
_NT = 25000
_N = _ND + _NT
_E = 400000
_H = 128
_L = 2

_NC = 2
_NS = 16
_NW = _NC * _NS

_E_PAD = 409600
_ACH = _E_PAD // _NW
_SUB = 2560
_TCH = _E_PAD // _NS

_N_A = 50016
_RSZ = 9472
_NR = 6
_N_X = _RSZ * _NR
_ACC_ROWS = 9728
_TRASH = _RSZ
_FR = _RSZ // _NS
_ZR = 76
_K = 256
_CMAX = 2832

_MESH = dict(core_axis_name="c", subcore_axis_name="s", num_cores=_NC,
             num_subcores=_NS)


def _proj_body(x_ref, w_ref, b_ref, o_ref):
  o_ref[...] = (
      jnp.dot(x_ref[...], w_ref[...], preferred_element_type=jnp.float32,
              precision=lax.Precision.HIGHEST)
      + b_ref[...][None, :]
  )


def _proj(x, w, b):
  rows, k = x.shape
  rb = 5000
  return pl.pallas_call(
      _proj_body,
      grid=(rows // rb,),
      in_specs=[
          pl.BlockSpec((rb, k), lambda i: (i, 0)),
          pl.BlockSpec((k, _H), lambda i: (0, 0)),
          pl.BlockSpec((_H,), lambda i: (0,)),
      ],
      out_specs=pl.BlockSpec((rb, _H), lambda i: (i, 0)),
      out_shape=jax.ShapeDtypeStruct((rows, _H), jnp.float32),
  )(x, w, b)


def _hw_body(x_ref, w_ref, a_ref, hw_ref, al_ref):
  h = jnp.dot(x_ref[...], w_ref[0], preferred_element_type=jnp.float32,
              precision=lax.Precision.HIGHEST)
  hw_ref[0] = h
  al_ref[0] = jnp.dot(h, a_ref[0], preferred_element_type=jnp.float32,
                      precision=lax.Precision.HIGHEST)


def _hw_alpha(x, w3, a3):
  rb = 5000
  return pl.pallas_call(
      _hw_body,
      grid=(3, _N // rb),
      in_specs=[
          pl.BlockSpec((rb, _H), lambda t, i: (i, 0)),
          pl.BlockSpec((1, _H, _H), lambda t, i: (t, 0, 0)),
          pl.BlockSpec((1, _H, 2), lambda t, i: (t, 0, 0)),
      ],
      out_specs=[
          pl.BlockSpec((1, rb, _H), lambda t, i: (t, i, 0)),
          pl.BlockSpec((1, rb, 2), lambda t, i: (t, i, 0)),
      ],
      out_shape=[
          jax.ShapeDtypeStruct((3, _N, _H), jnp.float32),
          jax.ShapeDtypeStruct((3, _N, 2), jnp.float32),
      ],
  )(x, w3, a3)


def _term_body(ea_ref, ae_ref, o_ref):
  o_ref[...] = jnp.dot(ea_ref[...], ae_ref[...],
                       preferred_element_type=jnp.float32,
                       precision=lax.Precision.HIGHEST)


def _edge_terms(edge_attr, a_edge):
  eb = 8000
  de = edge_attr.shape[1]
  return pl.pallas_call(
      _term_body,
      grid=(_E // eb,),
      in_specs=[
          pl.BlockSpec((eb, de), lambda i: (i, 0)),
          pl.BlockSpec((de, _L), lambda i: (0, 0)),
      ],
      out_specs=pl.BlockSpec((eb, _L), lambda i: (i, 0)),
      out_shape=jax.ShapeDtypeStruct((_E, _L), jnp.float32),
  )(edge_attr, a_edge.T)


def _make_phase_a(has_term):
  mesh = plsc.VectorSubcoreMesh(**_MESH)
  nsub = _ACH // _SUB
  nv = _SUB // 16

  scratch = [
      pltpu.VMEM_SHARED((_N_A,), jnp.float32),
      pltpu.VMEM_SHARED((_N_A,), jnp.float32),
      pltpu.VMEM((_N_A,), jnp.float32),
      pltpu.VMEM((_N_A,), jnp.float32),
      pltpu.VMEM((16,), jnp.float32),
      pltpu.VMEM((_SUB,), jnp.int32),
      pltpu.VMEM((_SUB,), jnp.int32),
      pltpu.VMEM((_SUB,), jnp.float32),
      pltpu.VMEM((_SUB,), jnp.float32),
  ]

  def body(as_hbm, ad_hbm, m_hbm, src_hbm, dst_hbm, term_hbm, e_hbm,
           as_sh, ad_sh, as_v, ad_v, mv, sv, dv, tv, ev):
    cid = lax.axis_index("c")
    sid = lax.axis_index("s")
    wid = sid * _NC + cid

    @pl.when(sid == 0)
    def _stage():
      pltpu.sync_copy(as_hbm, as_sh)
      pltpu.sync_copy(ad_hbm, ad_sh)

    pltpu.sync_copy(m_hbm, mv)
    plsc.subcore_barrier()
    pltpu.sync_copy(as_sh, as_v)
    pltpu.sync_copy(ad_sh, ad_v)
    m16 = mv[...]

    def sub(su, _):
      base = wid * _ACH + su * _SUB
      pltpu.sync_copy(src_hbm.at[pl.ds(base, _SUB)], sv)
      pltpu.sync_copy(dst_hbm.at[pl.ds(base, _SUB)], dv)
      if has_term:
        pltpu.sync_copy(term_hbm.at[pl.ds(base, _SUB)], tv)

      def vec(v, _):
        s16 = sv[pl.ds(v * 16, 16)]
        d16 = dv[pl.ds(v * 16, 16)]
        lg = plsc.load_gather(as_v, [s16]) + plsc.load_gather(ad_v, [d16])
        if has_term:
          lg = lg + tv[pl.ds(v * 16, 16)]
        lr = jnp.where(lg >= 0.0, lg, lg * jnp.float32(0.2))
        ev[pl.ds(v * 16, 16)] = jnp.exp(lr - m16)
        return 0

      lax.fori_loop(0, nv, vec, 0)
      pltpu.sync_copy(ev, e_hbm.at[pl.ds(base, _SUB)])
      return 0

    lax.fori_loop(0, nsub, sub, 0)

  return pl.kernel(
      body,
      out_type=jax.ShapeDtypeStruct((_E_PAD,), jnp.float32),
      mesh=mesh,
      scratch_types=scratch,
      compiler_params=pltpu.CompilerParams(needs_layout_passes=False),
  )


_phase_a_term = _make_phase_a(True)
_phase_a_noterm = _make_phase_a(False)


def _make_phase_b():
  mesh = plsc.VectorSubcoreMesh(**_MESH)
  nsub = _TCH // _SUB
  nv = _SUB // 16

  scratch = [
      pltpu.VMEM_SHARED((_ACC_ROWS, _H), jnp.float32),
      pltpu.VMEM_SHARED((_ACC_ROWS,), jnp.float32),
      pltpu.VMEM((608,), jnp.float32),
      pltpu.VMEM((_SUB,), jnp.int32),
      pltpu.VMEM((_SUB,), jnp.int32),
      pltpu.VMEM((_SUB,), jnp.float32),
      pltpu.VMEM((_CMAX,), jnp.int32),
      pltpu.VMEM((_CMAX,), jnp.int32),
      pltpu.VMEM((_CMAX,), jnp.float32),
      pltpu.VMEM((_K,), jnp.int32),
      pltpu.VMEM((_K, _H), jnp.float32),
      pltpu.SemaphoreType.DMA,
  ]

  def body(hw0, hw1, hw2, s0, d0, e0, s1, d1, e1, s2, d2, e2,
           acc0, acc1, acc2, den0, den1, den2,
           acc_sh, den_sh, zden, sv, dv, evv, sC, dC, eC,
           dBlk, rows, gsem):
    cid = lax.axis_index("c")
    sid = lax.axis_index("s")
    z16f = jnp.zeros((16,), jnp.float32)
    z16i = jnp.zeros((16,), jnp.int32)
    t16i = jnp.full((16,), _TRASH, jnp.int32)

    for i in range(608 // 16):
      zden[pl.ds(i * 16, 16)] = z16f

    def flush_block(hw, off, n_idx_stage):
      for i in range(_K // 16):
        dBlk[pl.ds(i * 16, 16)] = dC[pl.ds(off + i * 16, 16)]
      pltpu.async_copy(hw.at[sC.at[pl.ds(off, _K)]], rows, gsem).wait()

      def scale(i2, _):
        i = i2 * 2
        ev0 = plsc.load_gather(eC, [jnp.full((16,), off + i, jnp.int32)])
        ev1 = plsc.load_gather(eC, [jnp.full((16,), off + i + 1, jnp.int32)])
        for j in range(_H // 16):
          rows[i, pl.ds(j * 16, 16)] = rows[i, pl.ds(j * 16, 16)] * ev0
        for j in range(_H // 16):
          rows[i + 1, pl.ds(j * 16, 16)] = (
              rows[i + 1, pl.ds(j * 16, 16)] * ev1)
        return 0

      lax.fori_loop(0, _K // 2, scale, 0)

      pltpu.sync_copy(rows, acc_sh.at[dBlk], add=True)
      pltpu.sync_copy(eC.at[pl.ds(off, _K)], den_sh.at[dBlk], add=True)

    def one_type(lo, hw, ss, dd, ee, accT, denT):
      def zinit(i, _):
        for j in range(_H // 16):
          rows[i, pl.ds(j * 16, 16)] = z16f
        return 0

      lax.fori_loop(0, _ZR, zinit, 0)
      plsc.subcore_barrier()
      for z in range(608 // _ZR):
        pltpu.sync_copy(rows.at[pl.ds(0, _ZR), :],
                        acc_sh.at[pl.ds(sid * 608 + z * _ZR, _ZR), :])
      pltpu.sync_copy(zden, den_sh.at[pl.ds(sid * 608, 608)])
      plsc.subcore_barrier()

      def sub(su, cnt):
        base = sid * _TCH + su * _SUB
        pltpu.sync_copy(ss.at[pl.ds(base, _SUB)], sv)
        pltpu.sync_copy(dd.at[pl.ds(base, _SUB)], dv)
        pltpu.sync_copy(ee.at[pl.ds(base, _SUB)], evv)

        def vec(v, cnt):
          d16 = dv[pl.ds(v * 16, 16)]
          rel = d16 - lo
          msk = (rel >= 0) & (rel < _RSZ)
          plsc.store_compressed(sC.at[pl.ds(cnt, 16)],
                                sv[pl.ds(v * 16, 16)], mask=msk)
          plsc.store_compressed(dC.at[pl.ds(cnt, 16)], rel, mask=msk)
          plsc.store_compressed(eC.at[pl.ds(cnt, 16)],
                                evv[pl.ds(v * 16, 16)], mask=msk)
          return cnt + plsc.all_reduce_population_count(msk)[0]

        cnt = lax.fori_loop(0, nv, vec, cnt)

        nfull = lax.div(cnt, jnp.int32(_K))

        def blk(b, _):
          flush_block(hw, b * _K, 0)
          return 0

        lax.fori_loop(0, nfull, blk, 0)

        roff = nfull * _K
        for i in range(_K // 16):
          sC[pl.ds(i * 16, 16)] = sC[pl.ds(roff + i * 16, 16)]
          dC[pl.ds(i * 16, 16)] = dC[pl.ds(roff + i * 16, 16)]
          eC[pl.ds(i * 16, 16)] = eC[pl.ds(roff + i * 16, 16)]
        return cnt - roff

      cnt = lax.fori_loop(0, nsub, sub, jnp.int32(0))

      for i in range(_K // 16):
        sC[pl.ds(cnt + i * 16, 16)] = z16i
        dC[pl.ds(cnt + i * 16, 16)] = t16i
        eC[pl.ds(cnt + i * 16, 16)] = z16f

      @pl.when(cnt > 0)
      def _tail():
        flush_block(hw, 0, 0)

      plsc.subcore_barrier()

      pltpu.sync_copy(acc_sh.at[pl.ds(sid * _FR, _FR), :],
                      accT.at[pl.ds(lo + sid * _FR, _FR), :])

      pltpu.sync_copy(den_sh.at[pl.ds(sid * _FR, _FR)],
                      evv.at[pl.ds(0, _FR)])
      pltpu.sync_copy(evv.at[pl.ds(0, _FR)],
                      denT.at[pl.ds(lo + sid * _FR, _FR)])

    def one_range(r, _):
      lo = (cid * (_NR // _NC) + r) * _RSZ
      one_type(lo, hw0, s0, d0, e0, acc0, den0)
      one_type(lo, hw1, s1, d1, e1, acc1, den1)
      one_type(lo, hw2, s2, d2, e2, acc2, den2)
      return 0

    lax.fori_loop(0, _NR // _NC, one_range, 0)

  return pl.kernel(
      body,
      out_type=(
          jax.ShapeDtypeStruct((_N_X, _H), jnp.float32),
          jax.ShapeDtypeStruct((_N_X, _H), jnp.float32),
          jax.ShapeDtypeStruct((_N_X, _H), jnp.float32),
          jax.ShapeDtypeStruct((_N_X,), jnp.float32),
          jax.ShapeDtypeStruct((_N_X,), jnp.float32),
          jax.ShapeDtypeStruct((_N_X,), jnp.float32),
      ),
      mesh=mesh,
      scratch_types=scratch,
      compiler_params=pltpu.CompilerParams(needs_layout_passes=False),
  )


_phase_b = _make_phase_b()


def _fin_body(a0_ref, a1_ref, a2_ref, d0_ref, d1_ref, d2_ref, o_ref):
  acc = None
  for a_ref, d_ref in ((a0_ref, d0_ref), (a1_ref, d1_ref), (a2_ref, d2_ref)):
    o = a_ref[...] / (d_ref[...] + jnp.float32(1e-16))
    o = jnp.where(o > 0.0, o, jnp.exp(o) - jnp.float32(1.0))
    acc = o if acc is None else acc + o
  o_ref[...] = acc * jnp.float32(1.0 / 3.0)


def _finalize(a0, a1, a2, d0, d1, d2):
  rb = 3552
  return pl.pallas_call(
      _fin_body,
      grid=(_N_X // rb,),
      in_specs=[
          pl.BlockSpec((rb, _H), lambda i: (i, 0)),
          pl.BlockSpec((rb, _H), lambda i: (i, 0)),
          pl.BlockSpec((rb, _H), lambda i: (i, 0)),
          pl.BlockSpec((rb, 1), lambda i: (i, 0)),
          pl.BlockSpec((rb, 1), lambda i: (i, 0)),
          pl.BlockSpec((rb, 1), lambda i: (i, 0)),
      ],
      out_specs=pl.BlockSpec((rb, _H), lambda i: (i, 0)),
      out_shape=jax.ShapeDtypeStruct((_N_X, _H), jnp.float32),
  )(a0, a1, a2, d0[:, None], d1[:, None], d2[:, None])


def _pad_edges(ei):
  src = jnp.pad(ei[0], (0, _E_PAD - _E))
  dst = jnp.pad(ei[1], (0, _E_PAD - _E), constant_values=_N)
  return src, dst


@jax.jit
def _run(drug_x, target_x, edge_attr_dd, W_drug, b_drug, W_target, b_target,
         W_gat, a_src, a_dst, a_edge, edge_index_dd, edge_index_dt,
         edge_index_tt):
  xd = _proj(drug_x, W_drug, b_drug)
  xt = _proj(target_x, W_target, b_target)
  x = jnp.concatenate([xd, xt], axis=0)

  terms = _edge_terms(edge_attr_dd, a_edge)

  edges = [_pad_edges(edge_index_dd), _pad_edges(edge_index_dt),
           _pad_edges(edge_index_tt)]

  for l in range(_L):
    a3 = jnp.stack([a_src[l], a_dst[l]], axis=-1)
    hw, al = _hw_alpha(x, W_gat[l], a3)
    term = jnp.pad(terms[:, l], (0, _E_PAD - _E))
    es = []
    for t in range(3):
      a_s = jnp.pad(al[t, :, 0], (0, _N_A - _N))
      a_d = jnp.pad(al[t, :, 1], (0, _N_A - _N))
      m = jnp.max(al[t, :, 0]) + jnp.max(al[t, :, 1])
      if t == 0:
        m = m + jnp.max(terms[:, l])
      m = jnp.maximum(m, 0.0)
      m16 = jnp.full((16,), m, jnp.float32)
      src, dst = edges[t]
      if t == 0:
        e = _phase_a_term(a_s, a_d, m16, src, dst, term)
      else:
        e = _phase_a_noterm(a_s, a_d, m16, src, dst, term)
      es.append(e)

    a0, a1, a2, d0, d1, d2 = _phase_b(
        hw[0], hw[1], hw[2],
        edges[0][0], edges[0][1], es[0],
        edges[1][0], edges[1][1], es[1],
        edges[2][0], edges[2][1], es[2])
    xn = _finalize(a0, a1, a2, d0, d1, d2)
    x = xn[:_N]

  return x[:_ND], x[_ND:]


def kernel(drug_x, target_x, edge_attr_dd, W_drug, b_drug, W_target,
           b_target, W_gat, a_src, a_dst, a_edge, edge_index_dd,
           edge_index_dt, edge_index_tt):
  return _run(drug_x, target_x, edge_attr_dd, W_drug, b_drug, W_target,
              b_target, W_gat, a_src, a_dst, a_edge, edge_index_dd,
              edge_index_dt, edge_index_tt)
